# baseline (speedup 1.0000x reference)
import functools

import jax
import jax.numpy as jnp
from jax import lax
from jax.experimental import pallas as pl
from jax.experimental.pallas import tpu as pltpu
from jax.experimental.pallas import tpu_sc as plsc

_NUM_SAMPLES = 32


def kernel(x):
    C, T, H, W = x.shape
    n = _NUM_SAMPLES
    hch = 64
    nch = H // hch
    nbuf = 6
    lag = 3
    rows = C * n
    nw = 32
    per_w = rows // nw
    total = per_w * nch
    assert per_w * nw == rows and nch * hch == H

    mesh = plsc.VectorSubcoreMesh(core_axis_name="c", subcore_axis_name="s")

    x2 = x.reshape(C * T * H, W)

    @functools.partial(
        pl.kernel,
        mesh=mesh,
        out_type=jax.ShapeDtypeStruct((rows * H, W), x.dtype),
        scratch_types=[pltpu.VMEM((hch, W), x.dtype) for _ in range(nbuf)]
        + [pltpu.SemaphoreType.DMA] * (2 * nbuf),
    )
    def k(x_hbm, out_hbm, *scratch):
        vbuf = scratch[:nbuf]
        gsem, ssem = scratch[nbuf : 2 * nbuf], scratch[2 * nbuf :]
        wid = lax.axis_index("s") * 2 + lax.axis_index("c")

        def src_dst(q):
            frame, ch = q // nch, q % nch
            r = wid * per_w + frame
            c = r // n
            t = r % n
            tsrc = (t * (T - 1)) // (n - 1)
            src = x_hbm.at[pl.ds((c * T + tsrc) * H + ch * hch, hch), :]
            dst = out_hbm.at[pl.ds(r * H + ch * hch, hch), :]
            return src, dst

        def start_gather(q, b):
            src, _ = src_dst(q)
            pltpu.make_async_copy(src, vbuf[b], gsem[b]).start()

        waited = set()
        for q in range(min(nbuf, total)):
            start_gather(q, q % nbuf)
        for q in range(total):
            b = q % nbuf
            src, dst = src_dst(q)
            pltpu.make_async_copy(src, vbuf[b], gsem[b]).wait()
            pltpu.make_async_copy(vbuf[b], dst, ssem[b]).start()
            p = q - lag
            if p >= 0 and p + nbuf < total:
                pb = p % nbuf
                _, pdst = src_dst(p)
                pltpu.make_async_copy(vbuf[pb], pdst, ssem[pb]).wait()
                waited.add(p)
                start_gather(p + nbuf, pb)
        for q in range(total):
            if q not in waited:
                b = q % nbuf
                _, dst = src_dst(q)
                pltpu.make_async_copy(vbuf[b], dst, ssem[b]).wait()

    return k(x2).reshape(C, n, H, W)


# speedup vs baseline: 1.0059x; 1.0059x over previous
import functools

import jax
import jax.numpy as jnp
from jax import lax
from jax.experimental import pallas as pl
from jax.experimental.pallas import tpu as pltpu
from jax.experimental.pallas import tpu_sc as plsc

_NUM_SAMPLES = 32


def kernel(x):
    C, T, H, W = x.shape
    n = _NUM_SAMPLES
    hch = 64
    nch = H // hch
    nbuf = 6
    lag = 3
    rows = C * n
    nw = 32
    per_w = rows // nw
    total = per_w * nch
    assert per_w * nw == rows and nch * hch == H

    mesh = plsc.VectorSubcoreMesh(core_axis_name="c", subcore_axis_name="s")

    x2 = x.reshape(C * T * H, W)

    @functools.partial(
        pl.kernel,
        mesh=mesh,
        out_type=jax.ShapeDtypeStruct((rows * H, W), x.dtype),
        scratch_types=[pltpu.VMEM((hch, W), x.dtype) for _ in range(nbuf)]
        + [pltpu.SemaphoreType.DMA] * (2 * nbuf),
    )
    def k(x_hbm, out_hbm, *scratch):
        vbuf = scratch[:nbuf]
        gsem, ssem = scratch[nbuf : 2 * nbuf], scratch[2 * nbuf :]
        wid = lax.axis_index("c") * 16 + lax.axis_index("s")

        def src_dst(q):
            frame, ch = q // nch, q % nch
            r = wid * per_w + frame
            c = r // n
            t = r % n
            tsrc = (t * (T - 1)) // (n - 1)
            src = x_hbm.at[pl.ds((c * T + tsrc) * H + ch * hch, hch), :]
            dst = out_hbm.at[pl.ds(r * H + ch * hch, hch), :]
            return src, dst

        def start_gather(q, b):
            src, _ = src_dst(q)
            pltpu.make_async_copy(src, vbuf[b], gsem[b]).start()

        waited = set()
        for q in range(min(nbuf, total)):
            start_gather(q, q % nbuf)
        for q in range(total):
            b = q % nbuf
            src, dst = src_dst(q)
            pltpu.make_async_copy(src, vbuf[b], gsem[b]).wait()
            pltpu.make_async_copy(vbuf[b], dst, ssem[b]).start()
            p = q - lag
            if p >= 0 and p + nbuf < total:
                pb = p % nbuf
                _, pdst = src_dst(p)
                pltpu.make_async_copy(vbuf[pb], pdst, ssem[pb]).wait()
                waited.add(p)
                start_gather(p + nbuf, pb)
        for q in range(total):
            if q not in waited:
                b = q % nbuf
                _, dst = src_dst(q)
                pltpu.make_async_copy(vbuf[b], dst, ssem[b]).wait()

    return k(x2).reshape(C, n, H, W)


# speedup vs baseline: 1.0122x; 1.0063x over previous
import functools

import jax
from jax import lax
from jax.experimental import pallas as pl
from jax.experimental.pallas import tpu as pltpu
from jax.experimental.pallas import tpu_sc as plsc

_NUM_SAMPLES = 32


def kernel(x):
    C, T, H, W = x.shape
    n = _NUM_SAMPLES
    hch = 128
    nch = H // hch
    nbuf = 3
    rows = C * n
    nw = 32
    per_w = rows // nw
    total = per_w * nch
    assert per_w * nw == rows and nch * hch == H

    mesh = plsc.VectorSubcoreMesh(core_axis_name="c", subcore_axis_name="s")

    x2 = x.reshape(C * T * H, W)

    @functools.partial(
        pl.kernel,
        mesh=mesh,
        out_type=jax.ShapeDtypeStruct((rows * H, W), x.dtype),
        scratch_types=[pltpu.VMEM((hch, W), x.dtype) for _ in range(nbuf)]
        + [pltpu.SemaphoreType.DMA] * (2 * nbuf),
    )
    def k(x_hbm, out_hbm, *scratch):
        vbuf = scratch[:nbuf]
        gsem, ssem = scratch[nbuf : 2 * nbuf], scratch[2 * nbuf :]
        wid = lax.axis_index("s") * 2 + lax.axis_index("c")

        def src_dst(q):
            frame, ch = q // nch, q % nch
            r = wid * per_w + frame
            c = r // n
            t = r % n
            tsrc = (t * (T - 1)) // (n - 1)
            src = x_hbm.at[pl.ds((c * T + tsrc) * H + ch * hch, hch), :]
            dst = out_hbm.at[pl.ds(r * H + ch * hch, hch), :]
            return src, dst

        def start_gather(q, b):
            src, _ = src_dst(q)
            pltpu.make_async_copy(src, vbuf[b], gsem[b]).start()

        for q in range(min(nbuf, total)):
            start_gather(q, q % nbuf)
        for q in range(total):
            b = q % nbuf
            src, dst = src_dst(q)
            pltpu.make_async_copy(src, vbuf[b], gsem[b]).wait()
            scat = pltpu.make_async_copy(vbuf[b], dst, ssem[b])
            scat.start()
            if q + nbuf < total:
                scat.wait()
                start_gather(q + nbuf, b)
        for q in range(max(0, total - nbuf), total):
            b = q % nbuf
            _, dst = src_dst(q)
            pltpu.make_async_copy(vbuf[b], dst, ssem[b]).wait()

    return k(x2).reshape(C, n, H, W)
